# baseline (speedup 1.0000x reference)
import functools

import jax
import jax.numpy as jnp
from jax.experimental import pallas as pl
from jax.experimental.pallas import tpu as pltpu


def _fused_body(x_ref, w_ref, a_ref, b_ref, o_ref, s_ref, *, n_rel, n):
    m = pl.program_id(0)
    k = pl.program_id(1)

    @pl.when((m == 0) & (k == 0))
    def _supports():
        x = x_ref[...].astype(jnp.bfloat16)
        for r in range(n_rel):
            w = (w_ref[r] * (1.0 / n_rel)).astype(jnp.bfloat16)
            s_ref[r * n:(r + 1) * n, :] = jnp.dot(
                x, w, preferred_element_type=jnp.float32
            ).astype(jnp.bfloat16)

    a = a_ref[0].astype(jnp.bfloat16)
    s = s_ref[pl.ds(k * n, n), :]
    acc = jnp.dot(a, s, preferred_element_type=jnp.float32)

    @pl.when(k == 0)
    def _init():
        o_ref[...] = acc + b_ref[...]

    @pl.when(k > 0)
    def _accum():
        o_ref[...] += acc


def kernel(input_, adjs, adj_weight, bias):
    n_rel, n, _ = adjs.shape
    in_f = input_.shape[1]
    out_f = adj_weight.shape[2]
    bm = min(256, n)

    bias2 = bias.reshape(1, out_f)
    out = pl.pallas_call(
        functools.partial(_fused_body, n_rel=n_rel, n=n),
        grid=(n // bm, n_rel),
        in_specs=[
            pl.BlockSpec((n, in_f), lambda m, k: (0, 0)),
            pl.BlockSpec((n_rel, in_f, out_f), lambda m, k: (0, 0, 0)),
            pl.BlockSpec((1, bm, n), lambda m, k: (k, m, 0)),
            pl.BlockSpec((1, out_f), lambda m, k: (0, 0)),
        ],
        out_specs=pl.BlockSpec((bm, out_f), lambda m, k: (m, 0)),
        out_shape=jax.ShapeDtypeStruct((n, out_f), jnp.float32),
        scratch_shapes=[pltpu.VMEM((n_rel * n, out_f), jnp.bfloat16)],
        compiler_params=pltpu.CompilerParams(
            dimension_semantics=("arbitrary", "arbitrary"),
        ),
    )(input_, adj_weight, adjs, bias2)
    return out


# speedup vs baseline: 1.1549x; 1.1549x over previous
import functools

import jax
import jax.numpy as jnp
from jax.experimental import pallas as pl
from jax.experimental.pallas import tpu as pltpu


def _fused_body(x_ref, w_ref, a0_ref, a1_ref, b_ref, o_ref, s_ref, *, n_rel, n):
    m = pl.program_id(0)
    k = pl.program_id(1)
    h = n // 2

    @pl.when((m == 0) & (k == 0))
    def _supports():
        x = x_ref[...].astype(jnp.bfloat16)
        for r in range(n_rel):
            w = (w_ref[r] * (1.0 / n_rel)).astype(jnp.bfloat16)
            s_ref[r * n:(r + 1) * n, :] = jnp.dot(
                x, w, preferred_element_type=jnp.float32
            ).astype(jnp.bfloat16)

    a0 = a0_ref[0].astype(jnp.bfloat16)
    a1 = a1_ref[0].astype(jnp.bfloat16)
    s0 = s_ref[pl.ds(k * n, h), :]
    s1 = s_ref[pl.ds(k * n + h, h), :]
    acc = jnp.dot(a0, s0, preferred_element_type=jnp.float32)
    acc += jnp.dot(a1, s1, preferred_element_type=jnp.float32)

    @pl.when(k == 0)
    def _init():
        o_ref[...] = acc + b_ref[...]

    @pl.when(k > 0)
    def _accum():
        o_ref[...] += acc


def kernel(input_, adjs, adj_weight, bias):
    n_rel, n, _ = adjs.shape
    in_f = input_.shape[1]
    out_f = adj_weight.shape[2]
    bm = min(512, n)

    bias2 = bias.reshape(1, out_f)
    out = pl.pallas_call(
        functools.partial(_fused_body, n_rel=n_rel, n=n),
        grid=(n // bm, n_rel),
        in_specs=[
            pl.BlockSpec((n, in_f), lambda m, k: (0, 0)),
            pl.BlockSpec((n_rel, in_f, out_f), lambda m, k: (0, 0, 0)),
            pl.BlockSpec((1, bm, n // 2), lambda m, k: (k, m, 0)),
            pl.BlockSpec((1, bm, n // 2), lambda m, k: (k, m, 1)),
            pl.BlockSpec((1, out_f), lambda m, k: (0, 0)),
        ],
        out_specs=pl.BlockSpec((bm, out_f), lambda m, k: (m, 0)),
        out_shape=jax.ShapeDtypeStruct((n, out_f), jnp.float32),
        scratch_shapes=[pltpu.VMEM((n_rel * n, out_f), jnp.bfloat16)],
        compiler_params=pltpu.CompilerParams(
            dimension_semantics=("arbitrary", "arbitrary"),
        ),
    )(input_, adj_weight, adjs, adjs, bias2)
    return out


# speedup vs baseline: 1.1641x; 1.0080x over previous
import functools

import jax
import jax.numpy as jnp
from jax.experimental import pallas as pl
from jax.experimental.pallas import tpu as pltpu


def _fused_body(x_ref, w_ref, a_ref, b_ref, o_ref, s_ref, *, n_rel, n):
    m = pl.program_id(0)
    k = pl.program_id(1)

    @pl.when((m == 0) & (k == 0))
    def _supports():
        x = x_ref[...].astype(jnp.bfloat16)
        for r in range(n_rel):
            w = (w_ref[r] * (1.0 / n_rel)).astype(jnp.bfloat16)
            s_ref[r * n:(r + 1) * n, :] = jnp.dot(
                x, w, preferred_element_type=jnp.float32
            ).astype(jnp.bfloat16)

    a = a_ref[0].astype(jnp.bfloat16)
    s = s_ref[pl.ds(k * n, n), :]
    acc = jnp.dot(a, s, preferred_element_type=jnp.float32)

    @pl.when(k == 0)
    def _init():
        o_ref[...] = acc + b_ref[...]

    @pl.when(k > 0)
    def _accum():
        o_ref[...] += acc


def kernel(input_, adjs, adj_weight, bias):
    n_rel, n, _ = adjs.shape
    in_f = input_.shape[1]
    out_f = adj_weight.shape[2]
    bm = min(512, n)

    bias2 = bias.reshape(1, out_f)
    out = pl.pallas_call(
        functools.partial(_fused_body, n_rel=n_rel, n=n),
        grid=(n // bm, n_rel),
        in_specs=[
            pl.BlockSpec((n, in_f), lambda m, k: (0, 0)),
            pl.BlockSpec((n_rel, in_f, out_f), lambda m, k: (0, 0, 0)),
            pl.BlockSpec((1, bm, n), lambda m, k: (k, m, 0)),
            pl.BlockSpec((1, out_f), lambda m, k: (0, 0)),
        ],
        out_specs=pl.BlockSpec((bm, out_f), lambda m, k: (m, 0)),
        out_shape=jax.ShapeDtypeStruct((n, out_f), jnp.float32),
        scratch_shapes=[pltpu.VMEM((n_rel * n, out_f), jnp.bfloat16)],
        compiler_params=pltpu.CompilerParams(
            dimension_semantics=("arbitrary", "arbitrary"),
        ),
    )(input_, adj_weight, adjs, bias2)
    return out


# speedup vs baseline: 1.1707x; 1.0056x over previous
import functools

import jax
import jax.numpy as jnp
from jax.experimental import pallas as pl
from jax.experimental.pallas import tpu as pltpu


def _fused_body(x_ref, w_ref, a_ref, b_ref, o_ref, s_ref, *, n_rel, n):
    m = pl.program_id(0)
    k = pl.program_id(1)

    @pl.when(m == 0)
    def _supports():
        x = x_ref[...].astype(jnp.bfloat16)
        w = (w_ref[k] * (1.0 / n_rel)).astype(jnp.bfloat16)
        s_ref[pl.ds(k * n, n), :] = jnp.dot(
            x, w, preferred_element_type=jnp.float32
        ).astype(jnp.bfloat16)

    a = a_ref[0].astype(jnp.bfloat16)
    s = s_ref[pl.ds(k * n, n), :]
    acc = jnp.dot(a, s, preferred_element_type=jnp.float32)

    @pl.when(k == 0)
    def _init():
        o_ref[...] = acc + b_ref[...]

    @pl.when(k > 0)
    def _accum():
        o_ref[...] += acc


def kernel(input_, adjs, adj_weight, bias):
    n_rel, n, _ = adjs.shape
    in_f = input_.shape[1]
    out_f = adj_weight.shape[2]
    bm = min(512, n)

    bias2 = bias.reshape(1, out_f)
    out = pl.pallas_call(
        functools.partial(_fused_body, n_rel=n_rel, n=n),
        grid=(n // bm, n_rel),
        in_specs=[
            pl.BlockSpec((n, in_f), lambda m, k: (0, 0)),
            pl.BlockSpec((n_rel, in_f, out_f), lambda m, k: (0, 0, 0)),
            pl.BlockSpec((1, bm, n), lambda m, k: (k, m, 0)),
            pl.BlockSpec((1, out_f), lambda m, k: (0, 0)),
        ],
        out_specs=pl.BlockSpec((bm, out_f), lambda m, k: (m, 0)),
        out_shape=jax.ShapeDtypeStruct((n, out_f), jnp.float32),
        scratch_shapes=[pltpu.VMEM((n_rel * n, out_f), jnp.bfloat16)],
        compiler_params=pltpu.CompilerParams(
            dimension_semantics=("arbitrary", "arbitrary"),
        ),
    )(input_, adj_weight, adjs, bias2)
    return out
